# Initial kernel scaffold; baseline (speedup 1.0000x reference)
import jax
import jax.numpy as jnp
from jax.experimental import pallas as pl


def kernel(x, edge_index, trans_row, trans_col, trans_value, W, b):
    raise NotImplementedError("write your pallas kernel here")


# speedup vs baseline: 30.9069x; 30.9069x over previous
import functools

import jax
import jax.numpy as jnp
from jax import lax
from jax.experimental import pallas as pl
from jax.experimental.pallas import tpu as pltpu
from jax.experimental.pallas import tpu_sc as plsc

NC, NS, L = 2, 16, 16
N = 10000
N_PAD = 10240
C = 128
E = 160000
M = 2500
M_PAD = 2560
NNZ_PAD = 7680
CH = 80

_MESH = plsc.VectorSubcoreMesh(
    core_axis_name="c", subcore_axis_name="s", num_cores=NC, num_subcores=NS
)

_MAGIC = 0x5F3759DF


def _rsqrt16(x):
    i = plsc.bitcast(x, jnp.int32)
    y = plsc.bitcast(_MAGIC - lax.shift_right_logical(i, 1), jnp.float32)
    for _ in range(3):
        y = y * (1.5 - 0.5 * x * y * y)
    return jnp.where(x > 0.5, y, 0.0)


def _norm_body(row_hbm, col_hbm, norm_hbm, deg_sh, dinv_sh, zb, ones,
               rowv, colv, degv, dfull, normv):
    c = lax.axis_index("c")
    s = lax.axis_index("s")
    eps = E // NS
    nsl = 640

    def zero16(i, _):
        zb[pl.ds(i * L, L)] = jnp.zeros((L,), jnp.float32)
        return _
    lax.fori_loop(0, nsl // L, zero16, None)
    for g in range(CH // L):
        ones[pl.ds(g * L, L)] = jnp.ones((L,), jnp.float32)
    pltpu.sync_copy(zb, deg_sh.at[pl.ds(s * nsl, nsl)])
    plsc.subcore_barrier()

    def deg_chunk(t, _):
        base = s * eps + t * CH
        pltpu.sync_copy(row_hbm.at[pl.ds(base, CH)], rowv)
        pltpu.sync_copy(ones, deg_sh.at[rowv], add=True)
        return _
    lax.fori_loop(0, eps // CH, deg_chunk, None)
    plsc.subcore_barrier()

    pltpu.sync_copy(deg_sh.at[pl.ds(s * nsl, nsl)], degv)

    def dinv16(g, _):
        degv[pl.ds(g * L, L)] = _rsqrt16(degv[pl.ds(g * L, L)])
        return _
    lax.fori_loop(0, nsl // L, dinv16, None)
    pltpu.sync_copy(degv, dinv_sh.at[pl.ds(s * nsl, nsl)])
    plsc.subcore_barrier()
    pltpu.sync_copy(dinv_sh, dfull)

    @pl.when(c == 0)
    def _():
        def norm_chunk(t, _):
            base = s * eps + t * CH
            pltpu.sync_copy(row_hbm.at[pl.ds(base, CH)], rowv)
            pltpu.sync_copy(col_hbm.at[pl.ds(base, CH)], colv)
            for g in range(CH // L):
                sl = pl.ds(g * L, L)
                dr = plsc.load_gather(dfull, [rowv[sl]])
                dc = plsc.load_gather(dfull, [colv[sl]])
                normv[sl] = -(dr * dc)
            pltpu.sync_copy(normv, norm_hbm.at[pl.ds(base, CH)])
            return _
        lax.fori_loop(0, eps // CH, norm_chunk, None)


_SC_PARAMS = pltpu.CompilerParams(needs_layout_passes=False)


@functools.partial(
    pl.kernel,
    out_type=jax.ShapeDtypeStruct((E,), jnp.float32),
    mesh=_MESH,
    compiler_params=_SC_PARAMS,
    scratch_types=[
        pltpu.VMEM_SHARED((NS * 640,), jnp.float32),
        pltpu.VMEM_SHARED((NS * 640,), jnp.float32),
        pltpu.VMEM((640,), jnp.float32),
        pltpu.VMEM((CH,), jnp.float32),
        pltpu.VMEM((CH,), jnp.int32),
        pltpu.VMEM((CH,), jnp.int32),
        pltpu.VMEM((640,), jnp.float32),
        pltpu.VMEM((NS * 640,), jnp.float32),
        pltpu.VMEM((CH,), jnp.float32),
    ],
)
def _norm_kernel(row_hbm, col_hbm, norm_hbm, *rest):
    _norm_body(row_hbm, col_hbm, norm_hbm, *rest)


def _make_scatter(n_acc, n_entries, n_out_rows, zrows, zcopies):
    eps = n_entries // NS
    assert eps % CH == 0 and zrows * zcopies * NS == n_acc

    def body(src_hbm, val_hbm, dst_hbm, srcidx_hbm, out_hbm,
             acc_sh, zb, rowv, colv, valv, rows):
        c = lax.axis_index("c")
        s = lax.axis_index("s")

        def zero_row(i, _):
            for j in range(C // L):
                zb[i, pl.ds(j * L, L)] = jnp.zeros((L,), jnp.float32)
            return _
        lax.fori_loop(0, zrows, zero_row, None)
        for t in range(zcopies):
            pltpu.sync_copy(zb, acc_sh.at[pl.ds(s * zrows * zcopies + t * zrows, zrows)])
        plsc.subcore_barrier()

        coff = c * N_PAD

        def chunk(t, _):
            base = s * eps + t * CH
            pltpu.sync_copy(dst_hbm.at[pl.ds(base, CH)], rowv)
            pltpu.sync_copy(srcidx_hbm.at[pl.ds(base, CH)], colv)
            pltpu.sync_copy(val_hbm.at[pl.ds(base, CH)], valv)
            for g in range(CH // L):
                sl = pl.ds(g * L, L)
                colv[sl] = colv[sl] + coff
            pltpu.sync_copy(src_hbm.at[colv], rows)

            def scale(i, _):
                nb = plsc.load_gather(valv, [jnp.full((L,), i, jnp.int32)])
                for j in range(C // L):
                    sl = (i, pl.ds(j * L, L))
                    rows[sl] = rows[sl] * nb
                return _
            lax.fori_loop(0, CH, scale, None)
            pltpu.sync_copy(rows, acc_sh.at[rowv], add=True)
            return _
        lax.fori_loop(0, eps // CH, chunk, None)
        plsc.subcore_barrier()

        for t in range(zcopies):
            r = s * zrows * zcopies + t * zrows
            pltpu.sync_copy(acc_sh.at[pl.ds(r, zrows)], zb)
            pltpu.sync_copy(zb, out_hbm.at[pl.ds(c * n_out_rows + r, zrows)])

    return pl.kernel(
        body,
        out_type=jax.ShapeDtypeStruct((NC * n_out_rows, C), jnp.float32),
        mesh=_MESH,
        compiler_params=_SC_PARAMS,
        scratch_types=[
            pltpu.VMEM_SHARED((n_acc, C), jnp.float32),
            pltpu.VMEM((zrows, C), jnp.float32),
            pltpu.VMEM((CH,), jnp.int32),
            pltpu.VMEM((CH,), jnp.int32),
            pltpu.VMEM((CH,), jnp.float32),
            pltpu.VMEM((CH, C), jnp.float32),
        ],
    )


_prop_kernel = _make_scatter(n_acc=N_PAD, n_entries=E, n_out_rows=N_PAD,
                             zrows=128, zcopies=5)
_pool_kernel = _make_scatter(n_acc=M_PAD, n_entries=NNZ_PAD, n_out_rows=M_PAD,
                             zrows=80, zcopies=2)


def _mm_body(x_ref, p1_ref, p2_ref, w_ref, b_ref, o_ref):
    w0 = w_ref[0] - w_ref[2]
    w1 = w_ref[1]
    w2 = 2.0 * w_ref[2]
    z = jnp.dot(x_ref[...], w0, preferred_element_type=jnp.float32)
    z = z + jnp.dot(p1_ref[...], w1, preferred_element_type=jnp.float32)
    z = z + jnp.dot(p2_ref[...], w2, preferred_element_type=jnp.float32)
    z = z + b_ref[...]
    o_ref[...] = jnp.where(z > 0, z, jnp.exp(z) - 1.0)


def _mm_call(xf, p1f, p2f, W, b):
    BN = xf.shape[0]
    blk = 1024
    grid = BN // blk
    row_spec = pl.BlockSpec((blk, C), lambda i: (i, 0))
    return pl.pallas_call(
        _mm_body,
        grid=(grid,),
        in_specs=[row_spec, row_spec, row_spec,
                  pl.BlockSpec((3, C, C), lambda i: (0, 0, 0)),
                  pl.BlockSpec((1, C), lambda i: (0, 0))],
        out_specs=row_spec,
        out_shape=jax.ShapeDtypeStruct((BN, C), jnp.float32),
    )(xf, p1f, p2f, W, b)


def kernel(x, edge_index, trans_row, trans_col, trans_value, W, b):
    B = x.shape[0]
    row = edge_index[0]
    col = edge_index[1]

    norm = _norm_kernel(row, col)

    xf = jnp.pad(x, ((0, 0), (0, N_PAD - N), (0, 0))).reshape(B * N_PAD, C)
    p1f = _prop_kernel(xf, norm, row, col)
    p2f = _prop_kernel(p1f, norm, row, col)

    hf = _mm_call(xf, p1f, p2f, W, b.reshape(1, C))

    pad = NNZ_PAD - trans_row.shape[0]
    tr = jnp.concatenate([trans_row, jnp.zeros((pad,), jnp.int32)])
    tc = jnp.concatenate([trans_col, jnp.zeros((pad,), jnp.int32)])
    tv = jnp.concatenate([trans_value, jnp.zeros((pad,), jnp.float32)])

    pooled = _pool_kernel(hf, tv, tr, tc)
    return pooled.reshape(B, M_PAD, C)[:, :M, :]


# speedup vs baseline: 81.9656x; 2.6520x over previous
import functools

import jax
import jax.numpy as jnp
from jax import lax
from jax.experimental import pallas as pl
from jax.experimental.pallas import tpu as pltpu
from jax.experimental.pallas import tpu_sc as plsc

NC, NS, L = 2, 16, 16
N = 10000
N_PAD = 10240
C = 128
E = 160000
M = 2500
M_PAD = 2560
NNZ_PAD = 7680

_MESH = plsc.VectorSubcoreMesh(
    core_axis_name="c", subcore_axis_name="s", num_cores=NC, num_subcores=NS
)
_SC_PARAMS = pltpu.CompilerParams(needs_layout_passes=False)

_MAGIC = 0x5F3759DF


def _rsqrt16(x):
    i = plsc.bitcast(x, jnp.int32)
    y = plsc.bitcast(_MAGIC - lax.shift_right_logical(i, 1), jnp.float32)
    for _ in range(3):
        y = y * (1.5 - 0.5 * x * y * y)
    return jnp.where(x > 0.5, y, 0.0)


def _copy16(dst, dst_off, src, src_off, n16, add=None):
    for g in range(n16):
        v = src[pl.ds(src_off + g * L, L)]
        if add is not None:
            v = v + add
        dst[pl.ds(dst_off + g * L, L)] = v


def _norm_body(row_hbm, col_hbm, norm_hbm, deg_sh, dinv_sh, zb, ones,
               rowm, colm, rv0, rv1, dfull, nbuf, sd0, sd1):
    c = lax.axis_index("c")
    s = lax.axis_index("s")
    eps = E // NS
    nsl = 640
    ch = 80
    nch = eps // ch

    def zero16(i, _):
        zb[pl.ds(i * L, L)] = jnp.zeros((L,), jnp.float32)
        return _
    lax.fori_loop(0, nsl // L, zero16, None)
    for g in range(ch // L):
        ones[pl.ds(g * L, L)] = jnp.ones((L,), jnp.float32)
    pltpu.sync_copy(zb, deg_sh.at[pl.ds(s * nsl, nsl)])
    pltpu.sync_copy(row_hbm.at[pl.ds(s * eps, eps)], rowm)
    pltpu.sync_copy(col_hbm.at[pl.ds(s * eps, eps)], colm)
    plsc.subcore_barrier()

    def dstart(t, rv, sem):
        _copy16(rv, 0, rowm, t * ch, ch // L)
        pltpu.async_copy(ones, deg_sh.at[rv], sem, add=True)

    def dwait(rv, sem):
        pltpu.make_async_copy(ones, deg_sh.at[rv], sem).wait()

    dstart(0, rv0, sd0)
    dstart(1, rv1, sd1)

    def deg_step(g, _):
        t0 = 2 * g
        dwait(rv0, sd0)
        dstart(t0, rv0, sd0)
        dwait(rv1, sd1)
        dstart(t0 + 1, rv1, sd1)
        return _
    lax.fori_loop(1, nch // 2, deg_step, None)
    dwait(rv0, sd0)
    dstart(nch - 1, rv0, sd0)
    dwait(rv0, sd0)
    dwait(rv1, sd1)
    plsc.subcore_barrier()

    pltpu.sync_copy(deg_sh.at[pl.ds(s * nsl, nsl)], zb)

    def dinv16(g, _):
        zb[pl.ds(g * L, L)] = _rsqrt16(zb[pl.ds(g * L, L)])
        return _
    lax.fori_loop(0, nsl // L, dinv16, None)
    pltpu.sync_copy(zb, dinv_sh.at[pl.ds(s * nsl, nsl)])
    plsc.subcore_barrier()
    pltpu.sync_copy(dinv_sh, dfull)

    @pl.when(c == 0)
    def _():
        def norm16(k, _):
            sl = pl.ds(k * L, L)
            dr = plsc.load_gather(dfull, [rowm[sl]])
            dc = plsc.load_gather(dfull, [colm[sl]])
            nbuf[sl] = -(dr * dc)
            return _
        lax.fori_loop(0, eps // L, norm16, None)
        pltpu.sync_copy(nbuf, norm_hbm.at[pl.ds(s * eps, eps)])


@functools.partial(
    pl.kernel,
    out_type=jax.ShapeDtypeStruct((E,), jnp.float32),
    mesh=_MESH,
    compiler_params=_SC_PARAMS,
    scratch_types=[
        pltpu.VMEM_SHARED((NS * 640,), jnp.float32),
        pltpu.VMEM_SHARED((NS * 640,), jnp.float32),
        pltpu.VMEM((640,), jnp.float32),
        pltpu.VMEM((80,), jnp.float32),
        pltpu.VMEM((E // NS,), jnp.int32),
        pltpu.VMEM((E // NS,), jnp.int32),
        pltpu.VMEM((80,), jnp.int32),
        pltpu.VMEM((80,), jnp.int32),
        pltpu.VMEM((NS * 640,), jnp.float32),
        pltpu.VMEM((E // NS,), jnp.float32),
        pltpu.SemaphoreType.DMA,
        pltpu.SemaphoreType.DMA,
    ],
)
def _norm_kernel(row_hbm, col_hbm, norm_hbm, *rest):
    _norm_body(row_hbm, col_hbm, norm_hbm, *rest)


def _pack_chunks(dst, srcidx, val, ch):
    q = dst.shape[0] // ch
    packed = jnp.concatenate([
        dst.reshape(q, ch),
        srcidx.reshape(q, ch),
        lax.bitcast_convert_type(val, jnp.int32).reshape(q, ch),
    ], axis=1)
    return jnp.pad(packed, ((0, 2), (0, 0))).reshape(-1)


def _make_scatter(n_acc, n_entries, n_out_rows, zrows, zcopies, ch):
    eps = n_entries // NS
    nch = eps // ch
    assert eps % ch == 0 and nch % 2 == 1 and nch >= 3 and ch % L == 0 and ch <= 128
    assert zrows * zcopies * NS == n_acc

    def body(src_hbm, packed_hbm, out_hbm, acc_sh, zb,
             p0, p1, rv0, rv1, cv0, cv1, vv0, vv1, rows0, rows1,
             sp0, sp1, sg0, sg1):
        c = lax.axis_index("c")
        s = lax.axis_index("s")
        coff = c * N_PAD

        def zero_row(i, _):
            for j in range(C // L):
                zb[i, pl.ds(j * L, L)] = jnp.zeros((L,), jnp.float32)
            return _
        lax.fori_loop(0, zrows, zero_row, None)
        for t in range(zcopies):
            pltpu.sync_copy(zb, acc_sh.at[pl.ds(s * zrows * zcopies + t * zrows, zrows)])
        plsc.subcore_barrier()

        def pstart(t, p, sem):
            pltpu.async_copy(
                packed_hbm.at[pl.ds((s * nch + t) * 3 * ch, 3 * ch)], p, sem)

        def pwait(t, p, sem):
            pltpu.make_async_copy(
                packed_hbm.at[pl.ds((s * nch + t) * 3 * ch, 3 * ch)], p, sem).wait()

        def gstart(t, p, sp, rv, cv, vv, rows, sg):
            pwait(t, p, sp)
            for g in range(ch // L):
                sl = pl.ds(g * L, L)
                rv[sl] = p[pl.ds(g * L, L)]
                cv[sl] = p[pl.ds(ch + g * L, L)] + coff
                vv[sl] = plsc.bitcast(p[pl.ds(2 * ch + g * L, L)], jnp.float32)
            pltpu.async_copy(src_hbm.at[cv], rows, sg)
            pstart(t + 2, p, sp)

        def gwait(cv, rows, sg):
            pltpu.make_async_copy(src_hbm.at[cv], rows, sg).wait()

        def scale(vv, rows):
            def srow(i, _):
                nb = plsc.load_gather(vv, [jnp.full((L,), i, jnp.int32)])
                for j in range(C // L):
                    sl = (i, pl.ds(j * L, L))
                    rows[sl] = rows[sl] * nb
                return _
            lax.fori_loop(0, ch, srow, None)

        pstart(0, p0, sp0)
        pstart(1, p1, sp1)
        gstart(0, p0, sp0, rv0, cv0, vv0, rows0, sg0)

        def step(g, _):
            t0 = 2 * g
            gstart(t0 + 1, p1, sp1, rv1, cv1, vv1, rows1, sg1)
            gwait(cv0, rows0, sg0)
            scale(vv0, rows0)
            pltpu.sync_copy(rows0, acc_sh.at[rv0], add=True)
            gstart(t0 + 2, p0, sp0, rv0, cv0, vv0, rows0, sg0)
            gwait(cv1, rows1, sg1)
            scale(vv1, rows1)
            pltpu.sync_copy(rows1, acc_sh.at[rv1], add=True)
            return _
        lax.fori_loop(0, (nch - 1) // 2, step, None)
        gwait(cv0, rows0, sg0)
        scale(vv0, rows0)
        pltpu.sync_copy(rows0, acc_sh.at[rv0], add=True)
        pwait(nch + 1, p0, sp0)
        pwait(nch, p1, sp1)
        plsc.subcore_barrier()

        for t in range(zcopies):
            r = s * zrows * zcopies + t * zrows
            pltpu.sync_copy(acc_sh.at[pl.ds(r, zrows)], zb)
            pltpu.sync_copy(zb, out_hbm.at[pl.ds(c * n_out_rows + r, zrows)])

    return pl.kernel(
        body,
        out_type=jax.ShapeDtypeStruct((NC * n_out_rows, C), jnp.float32),
        mesh=_MESH,
        compiler_params=_SC_PARAMS,
        scratch_types=[
            pltpu.VMEM_SHARED((n_acc, C), jnp.float32),
            pltpu.VMEM((zrows, C), jnp.float32),
            pltpu.VMEM((3 * ch,), jnp.int32),
            pltpu.VMEM((3 * ch,), jnp.int32),
            pltpu.VMEM((ch,), jnp.int32),
            pltpu.VMEM((ch,), jnp.int32),
            pltpu.VMEM((ch,), jnp.int32),
            pltpu.VMEM((ch,), jnp.int32),
            pltpu.VMEM((ch,), jnp.float32),
            pltpu.VMEM((ch,), jnp.float32),
            pltpu.VMEM((ch, C), jnp.float32),
            pltpu.VMEM((ch, C), jnp.float32),
            pltpu.SemaphoreType.DMA,
            pltpu.SemaphoreType.DMA,
            pltpu.SemaphoreType.DMA,
            pltpu.SemaphoreType.DMA,
        ],
    )


_prop_kernel = _make_scatter(n_acc=N_PAD, n_entries=E, n_out_rows=N_PAD,
                             zrows=64, zcopies=10, ch=80)
_pool_kernel = _make_scatter(n_acc=M_PAD, n_entries=NNZ_PAD, n_out_rows=M_PAD,
                             zrows=80, zcopies=2, ch=96)


def _mm_body(x_ref, p1_ref, p2_ref, w_ref, b_ref, o_ref):
    w0 = w_ref[0] - w_ref[2]
    w1 = w_ref[1]
    w2 = 2.0 * w_ref[2]
    z = jnp.dot(x_ref[...], w0, preferred_element_type=jnp.float32)
    z = z + jnp.dot(p1_ref[...], w1, preferred_element_type=jnp.float32)
    z = z + jnp.dot(p2_ref[...], w2, preferred_element_type=jnp.float32)
    z = z + b_ref[...]
    o_ref[...] = jnp.where(z > 0, z, jnp.exp(z) - 1.0)


def _mm_call(xf, p1f, p2f, W, b):
    BN = xf.shape[0]
    blk = 1024
    grid = BN // blk
    row_spec = pl.BlockSpec((blk, C), lambda i: (i, 0))
    return pl.pallas_call(
        _mm_body,
        grid=(grid,),
        in_specs=[row_spec, row_spec, row_spec,
                  pl.BlockSpec((3, C, C), lambda i: (0, 0, 0)),
                  pl.BlockSpec((1, C), lambda i: (0, 0))],
        out_specs=row_spec,
        out_shape=jax.ShapeDtypeStruct((BN, C), jnp.float32),
    )(xf, p1f, p2f, W, b)


def kernel(x, edge_index, trans_row, trans_col, trans_value, W, b):
    B = x.shape[0]
    row = edge_index[0]
    col = edge_index[1]

    norm = _norm_kernel(row, col)

    xf = jnp.pad(x, ((0, 0), (0, N_PAD - N), (0, 0))).reshape(B * N_PAD, C)
    epack = _pack_chunks(row, col, norm, 80)
    p1f = _prop_kernel(xf, epack)
    p2f = _prop_kernel(p1f, epack)

    hf = _mm_call(xf, p1f, p2f, W, b.reshape(1, C))

    pad = NNZ_PAD - trans_row.shape[0]
    tr = jnp.concatenate([trans_row, jnp.zeros((pad,), jnp.int32)])
    tc = jnp.concatenate([trans_col, jnp.zeros((pad,), jnp.int32)])
    tv = jnp.concatenate([trans_value, jnp.zeros((pad,), jnp.float32)])

    pooled = _pool_kernel(hf, _pack_chunks(tr, tc, tv, 96))
    return pooled.reshape(B, M_PAD, C)[:, :M, :]


# speedup vs baseline: 104.3621x; 1.2732x over previous
import functools

import jax
import jax.numpy as jnp
from jax import lax
from jax.experimental import pallas as pl
from jax.experimental.pallas import tpu as pltpu
from jax.experimental.pallas import tpu_sc as plsc

NC, NS, L = 2, 16, 16
N = 10000
N_PAD = 10240
C = 128
E = 160000
M = 2500
M_PAD = 2560
NNZ_PAD = 7680

_MESH = plsc.VectorSubcoreMesh(
    core_axis_name="c", subcore_axis_name="s", num_cores=NC, num_subcores=NS
)
_SC_PARAMS = pltpu.CompilerParams(needs_layout_passes=False)

_MAGIC = 0x5F3759DF


def _rsqrt16(x):
    i = plsc.bitcast(x, jnp.int32)
    y = plsc.bitcast(_MAGIC - lax.shift_right_logical(i, 1), jnp.float32)
    for _ in range(3):
        y = y * (1.5 - 0.5 * x * y * y)
    return jnp.where(x > 0.5, y, 0.0)


def _copy16(dst, dst_off, src, src_off, n16, add=None):
    for g in range(n16):
        v = src[pl.ds(src_off + g * L, L)]
        if add is not None:
            v = v + add
        dst[pl.ds(dst_off + g * L, L)] = v


def _norm_body(row_hbm, col_hbm, norm_hbm, deg_sh, dinv_sh, zb, ones,
               rowm, colm, rv0, rv1, dfull, nbuf, sd0, sd1):
    c = lax.axis_index("c")
    s = lax.axis_index("s")
    eps = E // NS
    nsl = 640
    ch = 80
    nch = eps // ch

    def zero16(i, _):
        zb[pl.ds(i * L, L)] = jnp.zeros((L,), jnp.float32)
        return _
    lax.fori_loop(0, nsl // L, zero16, None)
    for g in range(ch // L):
        ones[pl.ds(g * L, L)] = jnp.ones((L,), jnp.float32)
    pltpu.sync_copy(zb, deg_sh.at[pl.ds(s * nsl, nsl)])
    pltpu.sync_copy(row_hbm.at[pl.ds(s * eps, eps)], rowm)
    pltpu.sync_copy(col_hbm.at[pl.ds(s * eps, eps)], colm)
    plsc.subcore_barrier()

    def dstart(t, rv, sem):
        _copy16(rv, 0, rowm, t * ch, ch // L)
        pltpu.async_copy(ones, deg_sh.at[rv], sem, add=True)

    def dwait(rv, sem):
        pltpu.make_async_copy(ones, deg_sh.at[rv], sem).wait()

    dstart(0, rv0, sd0)
    dstart(1, rv1, sd1)

    def deg_step(g, _):
        t0 = 2 * g
        dwait(rv0, sd0)
        dstart(t0, rv0, sd0)
        dwait(rv1, sd1)
        dstart(t0 + 1, rv1, sd1)
        return _
    lax.fori_loop(1, nch // 2, deg_step, None)
    dwait(rv0, sd0)
    dstart(nch - 1, rv0, sd0)
    dwait(rv0, sd0)
    dwait(rv1, sd1)
    plsc.subcore_barrier()

    pltpu.sync_copy(deg_sh.at[pl.ds(s * nsl, nsl)], zb)

    def dinv16(g, _):
        zb[pl.ds(g * L, L)] = _rsqrt16(zb[pl.ds(g * L, L)])
        return _
    lax.fori_loop(0, nsl // L, dinv16, None)
    pltpu.sync_copy(zb, dinv_sh.at[pl.ds(s * nsl, nsl)])
    plsc.subcore_barrier()
    pltpu.sync_copy(dinv_sh, dfull)

    @pl.when(c == 0)
    def _():
        def norm16(k, _):
            sl = pl.ds(k * L, L)
            dr = plsc.load_gather(dfull, [rowm[sl]])
            dc = plsc.load_gather(dfull, [colm[sl]])
            nbuf[sl] = -(dr * dc)
            return _
        lax.fori_loop(0, eps // L, norm16, None)
        pltpu.sync_copy(nbuf, norm_hbm.at[pl.ds(s * eps, eps)])


@functools.partial(
    pl.kernel,
    out_type=jax.ShapeDtypeStruct((E,), jnp.float32),
    mesh=_MESH,
    compiler_params=_SC_PARAMS,
    scratch_types=[
        pltpu.VMEM_SHARED((NS * 640,), jnp.float32),
        pltpu.VMEM_SHARED((NS * 640,), jnp.float32),
        pltpu.VMEM((640,), jnp.float32),
        pltpu.VMEM((80,), jnp.float32),
        pltpu.VMEM((E // NS,), jnp.int32),
        pltpu.VMEM((E // NS,), jnp.int32),
        pltpu.VMEM((80,), jnp.int32),
        pltpu.VMEM((80,), jnp.int32),
        pltpu.VMEM((NS * 640,), jnp.float32),
        pltpu.VMEM((E // NS,), jnp.float32),
        pltpu.SemaphoreType.DMA,
        pltpu.SemaphoreType.DMA,
    ],
)
def _norm_kernel(row_hbm, col_hbm, norm_hbm, *rest):
    _norm_body(row_hbm, col_hbm, norm_hbm, *rest)


def _pack_chunks(dst, srcidx, val, ch):
    q = dst.shape[0] // ch
    packed = jnp.concatenate([
        dst.reshape(q, ch),
        srcidx.reshape(q, ch),
        lax.bitcast_convert_type(val, jnp.int32).reshape(q, ch),
    ], axis=1)
    return jnp.pad(packed, ((0, 3), (0, 0))).reshape(-1)


def _make_scatter(n_acc, n_entries, n_out_rows, zrows, zcopies, ch):
    eps = n_entries // NS
    nch = eps // ch
    assert eps % ch == 0 and (nch - 2) % 3 == 0 and nch >= 5
    assert ch % L == 0 and ch <= 128
    assert zrows * zcopies * NS == n_acc

    def body(src_hbm, packed_hbm, out_hbm, acc_sh, zb,
             p0, p1, p2, rv0, rv1, rv2, cv0, cv1, cv2, vv0, vv1, vv2,
             rows0, rows1, rows2, sp0, sp1, sp2, sg0, sg1, sg2, ss0, ss1, ss2):
        c = lax.axis_index("c")
        s = lax.axis_index("s")
        coff = c * N_PAD
        P = [(p0, sp0), (p1, sp1), (p2, sp2)]
        R = [(rv0, cv0, vv0, rows0, sg0, ss0),
             (rv1, cv1, vv1, rows1, sg1, ss1),
             (rv2, cv2, vv2, rows2, sg2, ss2)]

        def zero_row(i, _):
            for j in range(C // L):
                zb[i, pl.ds(j * L, L)] = jnp.zeros((L,), jnp.float32)
            return _
        lax.fori_loop(0, zrows, zero_row, None)
        for t in range(zcopies):
            pltpu.sync_copy(zb, acc_sh.at[pl.ds(s * zrows * zcopies + t * zrows, zrows)])
        plsc.subcore_barrier()

        def pslice(t):
            return packed_hbm.at[pl.ds((s * nch + t) * 3 * ch, 3 * ch)]

        def pstart(t, b):
            p, sp = P[b]
            pltpu.async_copy(pslice(t), p, sp)

        def pwait(t, b):
            p, sp = P[b]
            pltpu.make_async_copy(pslice(t), p, sp).wait()

        def gstart(t, b):
            pwait(t, b)
            p, _ = P[b]
            rv, cv, vv, rows, sg, _ = R[b]
            for g in range(ch // L):
                sl = pl.ds(g * L, L)
                rv[sl] = p[pl.ds(g * L, L)]
                cv[sl] = p[pl.ds(ch + g * L, L)] + coff
                vv[sl] = plsc.bitcast(p[pl.ds(2 * ch + g * L, L)], jnp.float32)
            pltpu.async_copy(src_hbm.at[cv], rows, sg)
            pstart(t + 2, (b + 2) % 3)

        def gwait(b):
            _, cv, _, rows, sg, _ = R[b]
            pltpu.make_async_copy(src_hbm.at[cv], rows, sg).wait()

        def scale(b):
            _, _, vv, rows, _, _ = R[b]

            @plsc.parallel_loop(0, ch, 1, unroll=4)
            def srow(i):
                nb = plsc.load_gather(vv, [jnp.full((L,), i, jnp.int32)])
                for j in range(C // L):
                    sl = (i, pl.ds(j * L, L))
                    rows[sl] = rows[sl] * nb

        def sstart(b):
            rv, _, _, rows, _, ss = R[b]
            pltpu.async_copy(rows, acc_sh.at[rv], ss, add=True)

        def swait(b):
            rv, _, _, rows, _, ss = R[b]
            pltpu.make_async_copy(rows, acc_sh.at[rv], ss).wait()

        pstart(0, 0)
        pstart(1, 1)
        gstart(0, 0)
        gstart(1, 1)
        gwait(0); scale(0); sstart(0); gstart(2, 2)

        def chunk(t, b, bprev):
            gwait(b); scale(b); sstart(b); swait(bprev); gstart(t + 2, (b + 2) % 3)

        def step(g, _):
            t0 = 3 * g + 1
            chunk(t0, 1, 0)
            chunk(t0 + 1, 2, 1)
            chunk(t0 + 2, 0, 2)
            return _
        lax.fori_loop(0, (nch - 2) // 3, step, None)
        b_last = (nch - 1) % 3
        gwait(b_last); scale(b_last); sstart(b_last); swait((nch - 2) % 3)
        swait(b_last)
        gwait(nch % 3)
        pwait(nch + 1, (nch + 1) % 3)
        pwait(nch + 2, (nch + 2) % 3)
        plsc.subcore_barrier()

        for t in range(zcopies):
            r = s * zrows * zcopies + t * zrows
            pltpu.sync_copy(acc_sh.at[pl.ds(r, zrows)], zb)
            pltpu.sync_copy(zb, out_hbm.at[pl.ds(c * n_out_rows + r, zrows)])

    return pl.kernel(
        body,
        out_type=jax.ShapeDtypeStruct((NC * n_out_rows, C), jnp.float32),
        mesh=_MESH,
        compiler_params=_SC_PARAMS,
        scratch_types=(
            [pltpu.VMEM_SHARED((n_acc, C), jnp.float32),
             pltpu.VMEM((zrows, C), jnp.float32)]
            + [pltpu.VMEM((3 * ch,), jnp.int32)] * 3
            + [pltpu.VMEM((ch,), jnp.int32)] * 3
            + [pltpu.VMEM((ch,), jnp.int32)] * 3
            + [pltpu.VMEM((ch,), jnp.float32)] * 3
            + [pltpu.VMEM((ch, C), jnp.float32)] * 3
            + [pltpu.SemaphoreType.DMA] * 9
        ),
    )


_prop_kernel = _make_scatter(n_acc=N_PAD, n_entries=E, n_out_rows=N_PAD,
                             zrows=64, zcopies=10, ch=80)
_pool_kernel = _make_scatter(n_acc=M_PAD, n_entries=NNZ_PAD, n_out_rows=M_PAD,
                             zrows=80, zcopies=2, ch=96)


def _mm_body(x_ref, p1_ref, p2_ref, w_ref, b_ref, o_ref):
    w0 = w_ref[0] - w_ref[2]
    w1 = w_ref[1]
    w2 = 2.0 * w_ref[2]
    z = jnp.dot(x_ref[...], w0, preferred_element_type=jnp.float32)
    z = z + jnp.dot(p1_ref[...], w1, preferred_element_type=jnp.float32)
    z = z + jnp.dot(p2_ref[...], w2, preferred_element_type=jnp.float32)
    z = z + b_ref[...]
    o_ref[...] = jnp.where(z > 0, z, jnp.exp(z) - 1.0)


def _mm_call(xf, p1f, p2f, W, b):
    BN = xf.shape[0]
    blk = 1024
    grid = BN // blk
    row_spec = pl.BlockSpec((blk, C), lambda i: (i, 0))
    return pl.pallas_call(
        _mm_body,
        grid=(grid,),
        in_specs=[row_spec, row_spec, row_spec,
                  pl.BlockSpec((3, C, C), lambda i: (0, 0, 0)),
                  pl.BlockSpec((1, C), lambda i: (0, 0))],
        out_specs=row_spec,
        out_shape=jax.ShapeDtypeStruct((BN, C), jnp.float32),
    )(xf, p1f, p2f, W, b)


def kernel(x, edge_index, trans_row, trans_col, trans_value, W, b):
    B = x.shape[0]
    row = edge_index[0]
    col = edge_index[1]

    norm = _norm_kernel(row, col)

    xf = jnp.pad(x, ((0, 0), (0, N_PAD - N), (0, 0))).reshape(B * N_PAD, C)
    epack = _pack_chunks(row, col, norm, 80)
    p1f = _prop_kernel(xf, epack)
    p2f = _prop_kernel(p1f, epack)

    hf = _mm_call(xf, p1f, p2f, W, b.reshape(1, C))

    pad = NNZ_PAD - trans_row.shape[0]
    tr = jnp.concatenate([trans_row, jnp.zeros((pad,), jnp.int32)])
    tc = jnp.concatenate([trans_col, jnp.zeros((pad,), jnp.int32)])
    tv = jnp.concatenate([trans_value, jnp.zeros((pad,), jnp.float32)])

    pooled = _pool_kernel(hf, _pack_chunks(tr, tc, tv, 96))
    return pooled.reshape(B, M_PAD, C)[:, :M, :]


# speedup vs baseline: 104.9888x; 1.0060x over previous
import functools

import jax
import jax.numpy as jnp
from jax import lax
from jax.experimental import pallas as pl
from jax.experimental.pallas import tpu as pltpu
from jax.experimental.pallas import tpu_sc as plsc

NC, NS, L = 2, 16, 16
N = 10000
N_PAD = 10240
C = 128
E = 160000
M = 2500
M_PAD = 2560
NNZ_PAD = 7680

_MESH = plsc.VectorSubcoreMesh(
    core_axis_name="c", subcore_axis_name="s", num_cores=NC, num_subcores=NS
)
_SC_PARAMS = pltpu.CompilerParams(needs_layout_passes=False)

_MAGIC = 0x5F3759DF


def _rsqrt16(x):
    i = plsc.bitcast(x, jnp.int32)
    y = plsc.bitcast(_MAGIC - lax.shift_right_logical(i, 1), jnp.float32)
    for _ in range(3):
        y = y * (1.5 - 0.5 * x * y * y)
    return jnp.where(x > 0.5, y, 0.0)


def _copy16(dst, dst_off, src, src_off, n16, add=None):
    for g in range(n16):
        v = src[pl.ds(src_off + g * L, L)]
        if add is not None:
            v = v + add
        dst[pl.ds(dst_off + g * L, L)] = v


def _norm_body(row_hbm, col_hbm, norm_hbm, deg_sh, dinv_sh, zb, ones,
               rowm, colm, rv0, rv1, dfull, nbuf, sd0, sd1):
    c = lax.axis_index("c")
    s = lax.axis_index("s")
    eps = E // NS
    nsl = 640
    ch = 80
    nch = eps // ch

    def zero16(i, _):
        zb[pl.ds(i * L, L)] = jnp.zeros((L,), jnp.float32)
        return _
    lax.fori_loop(0, nsl // L, zero16, None)
    for g in range(ch // L):
        ones[pl.ds(g * L, L)] = jnp.ones((L,), jnp.float32)
    pltpu.sync_copy(zb, deg_sh.at[pl.ds(s * nsl, nsl)])
    pltpu.sync_copy(row_hbm.at[pl.ds(s * eps, eps)], rowm)
    pltpu.sync_copy(col_hbm.at[pl.ds(s * eps, eps)], colm)
    plsc.subcore_barrier()

    def dstart(t, rv, sem):
        _copy16(rv, 0, rowm, t * ch, ch // L)
        pltpu.async_copy(ones, deg_sh.at[rv], sem, add=True)

    def dwait(rv, sem):
        pltpu.make_async_copy(ones, deg_sh.at[rv], sem).wait()

    dstart(0, rv0, sd0)
    dstart(1, rv1, sd1)

    def deg_step(g, _):
        t0 = 2 * g
        dwait(rv0, sd0)
        dstart(t0, rv0, sd0)
        dwait(rv1, sd1)
        dstart(t0 + 1, rv1, sd1)
        return _
    lax.fori_loop(1, nch // 2, deg_step, None)
    dwait(rv0, sd0)
    dstart(nch - 1, rv0, sd0)
    dwait(rv0, sd0)
    dwait(rv1, sd1)
    plsc.subcore_barrier()

    pltpu.sync_copy(deg_sh.at[pl.ds(s * nsl, nsl)], zb)

    def dinv16(g, _):
        zb[pl.ds(g * L, L)] = _rsqrt16(zb[pl.ds(g * L, L)])
        return _
    lax.fori_loop(0, nsl // L, dinv16, None)
    pltpu.sync_copy(zb, dinv_sh.at[pl.ds(s * nsl, nsl)])
    plsc.subcore_barrier()
    pltpu.sync_copy(dinv_sh, dfull)

    @pl.when(c == 0)
    def _():
        def norm16(k, _):
            sl = pl.ds(k * L, L)
            dr = plsc.load_gather(dfull, [rowm[sl]])
            dc = plsc.load_gather(dfull, [colm[sl]])
            nbuf[sl] = -(dr * dc)
            return _
        lax.fori_loop(0, eps // L, norm16, None)
        pltpu.sync_copy(nbuf, norm_hbm.at[pl.ds(s * eps, eps)])


@functools.partial(
    pl.kernel,
    out_type=jax.ShapeDtypeStruct((E,), jnp.float32),
    mesh=_MESH,
    compiler_params=_SC_PARAMS,
    scratch_types=[
        pltpu.VMEM_SHARED((NS * 640,), jnp.float32),
        pltpu.VMEM_SHARED((NS * 640,), jnp.float32),
        pltpu.VMEM((640,), jnp.float32),
        pltpu.VMEM((80,), jnp.float32),
        pltpu.VMEM((E // NS,), jnp.int32),
        pltpu.VMEM((E // NS,), jnp.int32),
        pltpu.VMEM((80,), jnp.int32),
        pltpu.VMEM((80,), jnp.int32),
        pltpu.VMEM((NS * 640,), jnp.float32),
        pltpu.VMEM((E // NS,), jnp.float32),
        pltpu.SemaphoreType.DMA,
        pltpu.SemaphoreType.DMA,
    ],
)
def _norm_kernel(row_hbm, col_hbm, norm_hbm, *rest):
    _norm_body(row_hbm, col_hbm, norm_hbm, *rest)


def _pack_chunks(dst, srcidx, val, ch):
    q = dst.shape[0] // ch
    packed = jnp.concatenate([
        dst.reshape(q, ch),
        srcidx.reshape(q, ch),
        lax.bitcast_convert_type(val, jnp.int32).reshape(q, ch),
    ], axis=1)
    return jnp.pad(packed, ((0, 3), (0, 0))).reshape(-1)


def _make_scatter(n_acc, n_entries, n_out_rows, ch):
    eps = n_entries // NS
    nch = eps // ch
    arows = n_acc // NS
    assert eps % ch == 0 and (nch - 2) % 3 == 0 and nch >= 5
    assert ch % L == 0 and ch <= 128
    assert arows % 80 == 0 and ch >= 80
    nb = arows // 80

    def body(src_hbm, packed_hbm, out_hbm, acc_sh,
             p0, p1, p2, rv0, rv1, rv2, cv0, cv1, cv2, vv0, vv1, vv2,
             rows0, rows1, rows2, sp0, sp1, sp2, sg0, sg1, sg2, ss0, ss1, ss2):
        c = lax.axis_index("c")
        s = lax.axis_index("s")
        coff = c * N_PAD
        P = [(p0, sp0), (p1, sp1), (p2, sp2)]
        R = [(rv0, cv0, vv0, rows0, sg0, ss0),
             (rv1, cv1, vv1, rows1, sg1, ss1),
             (rv2, cv2, vv2, rows2, sg2, ss2)]

        def zero_row(i, _):
            for j in range(C // L):
                rows0[i, pl.ds(j * L, L)] = jnp.zeros((L,), jnp.float32)
            return _
        lax.fori_loop(0, 80, zero_row, None)
        zsrc = rows0 if ch == 80 else rows0.at[pl.ds(0, 80)]
        for k in range(nb):
            pltpu.async_copy(zsrc, acc_sh.at[pl.ds(s * arows + k * 80, 80)], sg0)
        for k in range(nb):
            pltpu.make_async_copy(zsrc, acc_sh.at[pl.ds(s * arows + k * 80, 80)], sg0).wait()
        plsc.subcore_barrier()

        def pslice(t):
            return packed_hbm.at[pl.ds((s * nch + t) * 3 * ch, 3 * ch)]

        def pstart(t, b):
            p, sp = P[b]
            pltpu.async_copy(pslice(t), p, sp)

        def pwait(t, b):
            p, sp = P[b]
            pltpu.make_async_copy(pslice(t), p, sp).wait()

        def gstart(t, b):
            pwait(t, b)
            p, _ = P[b]
            rv, cv, vv, rows, sg, _ = R[b]
            for g in range(ch // L):
                sl = pl.ds(g * L, L)
                rv[sl] = p[pl.ds(g * L, L)]
                cv[sl] = p[pl.ds(ch + g * L, L)] + coff
                vv[sl] = plsc.bitcast(p[pl.ds(2 * ch + g * L, L)], jnp.float32)
            pltpu.async_copy(src_hbm.at[cv], rows, sg)
            pstart(t + 2, (b + 2) % 3)

        def gwait(b):
            _, cv, _, rows, sg, _ = R[b]
            pltpu.make_async_copy(src_hbm.at[cv], rows, sg).wait()

        def scale(b):
            _, _, vv, rows, _, _ = R[b]

            @plsc.parallel_loop(0, ch, 1, unroll=4)
            def srow(i):
                nb = plsc.load_gather(vv, [jnp.full((L,), i, jnp.int32)])
                for j in range(C // L):
                    sl = (i, pl.ds(j * L, L))
                    rows[sl] = rows[sl] * nb

        def sstart(b):
            rv, _, _, rows, _, ss = R[b]
            pltpu.async_copy(rows, acc_sh.at[rv], ss, add=True)

        def swait(b):
            rv, _, _, rows, _, ss = R[b]
            pltpu.make_async_copy(rows, acc_sh.at[rv], ss).wait()

        pstart(0, 0)
        pstart(1, 1)
        gstart(0, 0)
        gstart(1, 1)
        gwait(0); scale(0); sstart(0); gstart(2, 2)

        def chunk(t, b, bprev):
            gwait(b); scale(b); sstart(b); swait(bprev); gstart(t + 2, (b + 2) % 3)

        def step(g, _):
            t0 = 3 * g + 1
            chunk(t0, 1, 0)
            chunk(t0 + 1, 2, 1)
            chunk(t0 + 2, 0, 2)
            return _
        lax.fori_loop(0, (nch - 2) // 3, step, None)
        b_last = (nch - 1) % 3
        gwait(b_last); scale(b_last); sstart(b_last); swait((nch - 2) % 3)
        swait(b_last)
        gwait(nch % 3)
        pwait(nch + 1, (nch + 1) % 3)
        pwait(nch + 2, (nch + 2) % 3)
        plsc.subcore_barrier()

        r = s * arows
        pltpu.sync_copy(acc_sh.at[pl.ds(r, arows)],
                        out_hbm.at[pl.ds(c * n_out_rows + r, arows)])

    return pl.kernel(
        body,
        out_type=jax.ShapeDtypeStruct((NC * n_out_rows, C), jnp.float32),
        mesh=_MESH,
        compiler_params=_SC_PARAMS,
        scratch_types=(
            [pltpu.VMEM_SHARED((n_acc, C), jnp.float32)]
            + [pltpu.VMEM((3 * ch,), jnp.int32)] * 3
            + [pltpu.VMEM((ch,), jnp.int32)] * 3
            + [pltpu.VMEM((ch,), jnp.int32)] * 3
            + [pltpu.VMEM((ch,), jnp.float32)] * 3
            + [pltpu.VMEM((ch, C), jnp.float32)] * 3
            + [pltpu.SemaphoreType.DMA] * 9
        ),
    )


_prop_kernel = _make_scatter(n_acc=N_PAD, n_entries=E, n_out_rows=N_PAD, ch=80)
_pool_kernel = _make_scatter(n_acc=M_PAD, n_entries=NNZ_PAD, n_out_rows=M_PAD,
                             ch=96)


def _mm_body(x_ref, p1_ref, p2_ref, w_ref, b_ref, o_ref):
    w0 = w_ref[0] - w_ref[2]
    w1 = w_ref[1]
    w2 = 2.0 * w_ref[2]
    z = jnp.dot(x_ref[...], w0, preferred_element_type=jnp.float32)
    z = z + jnp.dot(p1_ref[...], w1, preferred_element_type=jnp.float32)
    z = z + jnp.dot(p2_ref[...], w2, preferred_element_type=jnp.float32)
    z = z + b_ref[...]
    o_ref[...] = jnp.where(z > 0, z, jnp.exp(z) - 1.0)


def _mm_call(xf, p1f, p2f, W, b):
    BN = xf.shape[0]
    blk = 1024
    grid = BN // blk
    row_spec = pl.BlockSpec((blk, C), lambda i: (i, 0))
    return pl.pallas_call(
        _mm_body,
        grid=(grid,),
        in_specs=[row_spec, row_spec, row_spec,
                  pl.BlockSpec((3, C, C), lambda i: (0, 0, 0)),
                  pl.BlockSpec((1, C), lambda i: (0, 0))],
        out_specs=row_spec,
        out_shape=jax.ShapeDtypeStruct((BN, C), jnp.float32),
    )(xf, p1f, p2f, W, b)


def kernel(x, edge_index, trans_row, trans_col, trans_value, W, b):
    B = x.shape[0]
    row = edge_index[0]
    col = edge_index[1]

    norm = _norm_kernel(row, col)

    xf = jnp.pad(x, ((0, 0), (0, N_PAD - N), (0, 0))).reshape(B * N_PAD, C)
    epack = _pack_chunks(row, col, norm, 80)
    p1f = _prop_kernel(xf, epack)
    p2f = _prop_kernel(p1f, epack)

    hf = _mm_call(xf, p1f, p2f, W, b.reshape(1, C))

    pad = NNZ_PAD - trans_row.shape[0]
    tr = jnp.concatenate([trans_row, jnp.zeros((pad,), jnp.int32)])
    tc = jnp.concatenate([trans_col, jnp.zeros((pad,), jnp.int32)])
    tv = jnp.concatenate([trans_value, jnp.zeros((pad,), jnp.float32)])

    pooled = _pool_kernel(hf, _pack_chunks(tr, tc, tv, 96))
    return pooled.reshape(B, M_PAD, C)[:, :M, :]
